# baseline (speedup 1.0000x reference)
import functools

import jax
import jax.numpy as jnp
from jax import lax
from jax.experimental import pallas as pl
from jax.experimental.pallas import tpu as pltpu
from jax.experimental.pallas import tpu_sc as plsc

_NC = 2
_NS = 16
_LANES = 16
_C = 64


def _sc_gather(td, ts, dstg, srcg):
    H2 = td.shape[1]
    E2 = dstg.shape[0]
    E = E2 // _NC
    nchunk = E // _C
    per_tile = nchunk // _NS

    mesh = plsc.VectorSubcoreMesh(core_axis_name="c", subcore_axis_name="s")

    @functools.partial(
        pl.kernel,
        out_type=[jax.ShapeDtypeStruct((E2, H2), jnp.float32)] * 2,
        mesh=mesh,
        scratch_types=[
            pltpu.VMEM((_C,), jnp.int32),
            pltpu.VMEM((_C,), jnp.int32),
            pltpu.VMEM((_C, H2), jnp.float32),
            pltpu.VMEM((_C, H2), jnp.float32),
            pltpu.SemaphoreType.DMA,
        ],
    )
    def k(td_hbm, ts_hbm, dstg_hbm, srcg_hbm, gd_hbm, gs_hbm,
          dst_v, src_v, bufd, bufs, sem):
        s_idx = lax.axis_index("s")
        c_idx = lax.axis_index("c")

        def _chunk(i, carry):
            cbase = c_idx * E + (s_idx + i * _NS) * _C
            sl = pl.ds(cbase, _C)
            pltpu.sync_copy(dstg_hbm.at[sl], dst_v)
            pltpu.sync_copy(srcg_hbm.at[sl], src_v)
            cp1 = pltpu.async_copy(td_hbm.at[dst_v], bufd, sem)
            cp2 = pltpu.async_copy(ts_hbm.at[src_v], bufs, sem)
            cp1.wait()
            cp2.wait()
            pltpu.sync_copy(bufd, gd_hbm.at[sl])
            pltpu.sync_copy(bufs, gs_hbm.at[sl])
            return carry

        lax.fori_loop(0, per_tile, _chunk, 0)

    return k(td, ts, dstg, srcg)


def _gate(gd, gs, ea, we, eb):
    R, H2 = gd.shape
    H = H2 // 2
    Ep = R // _NC
    BR = 8192
    nb = Ep // BR

    def body(gd_ref, gs_ref, ea_ref, we_ref, eb_ref, m_ref):
        et = (jnp.dot(ea_ref[...], we_ref[0],
                      preferred_element_type=jnp.float32) + eb_ref[0])
        z = gd_ref[...] + gs_ref[...] + et
        f = z[:, :H]
        s = z[:, H:]
        sp = jnp.maximum(s, 0.0) + jnp.log1p(jnp.exp(-jnp.abs(s)))
        m_ref[...] = jax.nn.sigmoid(f) * sp

    return pl.pallas_call(
        body,
        grid=(_NC * nb,),
        in_specs=[pl.BlockSpec((BR, H2), lambda i: (i, 0)),
                  pl.BlockSpec((BR, H2), lambda i: (i, 0)),
                  pl.BlockSpec((BR, ea.shape[1]), lambda i: (i % nb, 0)),
                  pl.BlockSpec((1,) + we.shape[1:], lambda i: (i // nb, 0, 0)),
                  pl.BlockSpec((1, 1, H2), lambda i: (i // nb, 0, 0))],
        out_specs=pl.BlockSpec((BR, H), lambda i: (i, 0)),
        out_shape=jax.ShapeDtypeStruct((R, H), jnp.float32),
    )(gd, gs, ea, we, eb)


def _node_tables(x, wd, ws):
    NP, D = x.shape
    BN = 2048
    H2 = wd.shape[2]

    def body(x_ref, wd_ref, ws_ref, td_ref, ts_ref):
        xb = x_ref[...]
        for c in range(_NC):
            td_ref[c] = jnp.dot(xb, wd_ref[c],
                                preferred_element_type=jnp.float32)
            ts_ref[c] = jnp.dot(xb, ws_ref[c],
                                preferred_element_type=jnp.float32)

    ospec = pl.BlockSpec((_NC, BN, H2), lambda i: (0, i, 0))
    return pl.pallas_call(
        body,
        grid=(NP // BN,),
        in_specs=[pl.BlockSpec((BN, D), lambda i: (i, 0)),
                  pl.BlockSpec(wd.shape, lambda i: (0, 0, 0)),
                  pl.BlockSpec(ws.shape, lambda i: (0, 0, 0))],
        out_specs=[ospec] * 2,
        out_shape=[jax.ShapeDtypeStruct((_NC, NP, H2), jnp.float32)] * 2,
    )(x, wd, ws)


def _mid(x, p, wd, ws):
    NP, D = x.shape
    BN = 2048
    H2 = wd.shape[2]
    H = p.shape[2]

    def body(x_ref, p_ref, wd_ref, ws_ref, y_ref, td_ref, ts_ref):
        agg = jnp.concatenate([p_ref[0], p_ref[1]], axis=1)
        y = jax.nn.silu(x_ref[...] + agg)
        y_ref[...] = y
        for c in range(_NC):
            td_ref[c] = jnp.dot(y, wd_ref[c],
                                preferred_element_type=jnp.float32)
            ts_ref[c] = jnp.dot(y, ws_ref[c],
                                preferred_element_type=jnp.float32)

    ospec = pl.BlockSpec((_NC, BN, H2), lambda i: (0, i, 0))
    return pl.pallas_call(
        body,
        grid=(NP // BN,),
        in_specs=[pl.BlockSpec((BN, D), lambda i: (i, 0)),
                  pl.BlockSpec((_NC, BN, H), lambda i: (0, i, 0)),
                  pl.BlockSpec(wd.shape, lambda i: (0, 0, 0)),
                  pl.BlockSpec(ws.shape, lambda i: (0, 0, 0))],
        out_specs=[pl.BlockSpec((BN, D), lambda i: (i, 0)),
                   ospec, ospec],
        out_shape=[jax.ShapeDtypeStruct((NP, D), jnp.float32),
                   jax.ShapeDtypeStruct((_NC, NP, H2), jnp.float32),
                   jax.ShapeDtypeStruct((_NC, NP, H2), jnp.float32)],
    )(x, p, wd, ws)


def _head(x, p, n_nodes, w1, b1, w2a, w2b, b2, w3, b3, w4, b4, w5, b5,
          w6, b6, w7, b7, g):
    N = n_nodes

    def body(x_ref, p_ref, w1_ref, b1_ref, w2a_ref, w2b_ref, b2_ref,
             w3_ref, b3_ref, w4_ref, b4_ref, w5_ref, b5_ref, w6_ref,
             b6_ref, w7_ref, b7_ref, g_ref, o_ref):
        agg = jnp.concatenate([p_ref[0], p_ref[1]], axis=1)
        x3 = jax.nn.silu(x_ref[...] + agg)
        t = jax.nn.silu(jnp.dot(x3, w1_ref[...],
                                preferred_element_type=jnp.float32)
                        + b1_ref[...])
        t = jax.nn.silu(jnp.dot(t, w1_ref[...],
                                preferred_element_type=jnp.float32)
                        + b1_ref[...])
        pooled = jnp.sum(t[:N], axis=0, keepdims=True) * jnp.float32(1.0 / N)
        h = jax.nn.silu(
            jnp.dot(pooled, w2a_ref[...], preferred_element_type=jnp.float32)
            + jnp.dot(g_ref[...], w2b_ref[...],
                      preferred_element_type=jnp.float32)
            + b2_ref[...])
        for w_ref, b_ref in ((w3_ref, b3_ref), (w4_ref, b4_ref),
                             (w5_ref, b5_ref), (w6_ref, b6_ref)):
            h = jax.nn.silu(jnp.dot(h, w_ref[...],
                                    preferred_element_type=jnp.float32)
                            + b_ref[...])
        o_ref[...] = (jnp.dot(h, w7_ref[...],
                              preferred_element_type=jnp.float32)
                      + b7_ref[...])

    args = (x, p, w1, b1, w2a, w2b, b2, w3, b3, w4, b4, w5, b5, w6, b6,
            w7, b7, g)
    specs = [pl.BlockSpec(a.shape, lambda i, n=a.ndim: (0,) * n)
             for a in args]
    return pl.pallas_call(
        body,
        grid=(1,),
        in_specs=specs,
        out_specs=pl.BlockSpec((1, 1), lambda i: (0, 0)),
        out_shape=jax.ShapeDtypeStruct((1, 1), jnp.float32),
    )(*args)


def kernel(x, edge_index, edge_attr, glob_attr, batch,
           Wf1, bf1, Ws1, bs1, Wf2, bf2, Ws2, bs2, Wf3, bf3, Ws3, bs3,
           W1, b1, W2, b2, W3, b3, W4, b4, W5, b5, W6, b6, W7, b7):
    N, D = x.shape
    E = edge_index.shape[1]
    H = D // _NC
    npad = ((N + 128 * _NS - 1) // (128 * _NS)) * 128 * _NS
    egrp = 4096
    epad = ((E + egrp - 1) // egrp) * egrp

    xp = jnp.pad(x, ((0, npad - N), (0, 0)))
    src = jnp.pad(edge_index[0], (0, epad - E), constant_values=npad - 1)
    dst = jnp.pad(edge_index[1], (0, epad - E), constant_values=npad - 1)
    eap = jnp.pad(edge_attr, ((0, epad - E), (0, 0)))
    dstg = jnp.concatenate([dst, dst + npad])
    srcg = jnp.concatenate([src, src + npad])

    def wsplit(Wf, Ws, bf, bs):
        def cat(rows):
            return jnp.stack([
                jnp.concatenate([Wf[rows, c * H:(c + 1) * H],
                                 Ws[rows, c * H:(c + 1) * H]], axis=1)
                for c in range(_NC)])
        wd = cat(slice(0, D))
        wsrc = cat(slice(D, 2 * D))
        we = cat(slice(2 * D, None))
        eb = jnp.stack([
            jnp.concatenate([bf[c * H:(c + 1) * H],
                             bs[c * H:(c + 1) * H]])[None, :]
            for c in range(_NC)])
        return wd, wsrc, we, eb

    wd1, wsrc1, we1, eb1 = wsplit(Wf1, Ws1, bf1, bs1)
    wd2, wsrc2, we2, eb2 = wsplit(Wf2, Ws2, bf2, bs2)
    wd3, wsrc3, we3, eb3 = wsplit(Wf3, Ws3, bf3, bs3)

    def conv(td, ts, we, eb):
        gd, gs = _sc_gather(td.reshape(_NC * npad, 2 * H),
                            ts.reshape(_NC * npad, 2 * H), dstg, srcg)
        m = _gate(gd, gs, eap, we, eb)
        return jnp.stack(
            [jax.ops.segment_sum(m[c * epad:(c + 1) * epad], dst,
                                 num_segments=npad)
             for c in range(_NC)])

    td, ts = _node_tables(xp, wd1, wsrc1)
    p = conv(td, ts, we1, eb1)
    x1, td, ts = _mid(xp, p, wd2, wsrc2)
    p = conv(td, ts, we2, eb2)
    x2, td, ts = _mid(x1, p, wd3, wsrc3)
    p = conv(td, ts, we3, eb3)

    w2a = W2[:D]
    w2b = W2[D:]
    return _head(x2, p, N, W1, b1[None, :], w2a, w2b, b2[None, :],
                 W3, b3[None, :], W4, b4[None, :], W5, b5[None, :],
                 W6, b6[None, :], W7, b7[None, :], glob_attr.reshape(1, -1))


# speedup vs baseline: 1.0233x; 1.0233x over previous
import functools

import jax
import jax.numpy as jnp
from jax import lax
from jax.experimental import pallas as pl
from jax.experimental.pallas import tpu as pltpu
from jax.experimental.pallas import tpu_sc as plsc

_NC = 2
_NS = 16
_LANES = 16
_C = 64


def _sc_gather(td, ts, dstg, srcg):
    H2 = td.shape[1]
    E2 = dstg.shape[0]
    E = E2 // _NC
    nchunk = E // _C
    per_tile = nchunk // _NS

    mesh = plsc.VectorSubcoreMesh(core_axis_name="c", subcore_axis_name="s")

    @functools.partial(
        pl.kernel,
        out_type=[jax.ShapeDtypeStruct((E2, H2), jnp.float32)] * 2,
        mesh=mesh,
        scratch_types=[
            pltpu.VMEM((_C,), jnp.int32),
            pltpu.VMEM((_C,), jnp.int32),
            pltpu.VMEM((_C, H2), jnp.float32),
            pltpu.VMEM((_C, H2), jnp.float32),
            pltpu.SemaphoreType.DMA,
            pltpu.SemaphoreType.DMA,
        ],
    )
    def k(td_hbm, ts_hbm, dstg_hbm, srcg_hbm, gd_hbm, gs_hbm,
          dst_v, src_v, bufd, bufs, sem, sem2):
        s_idx = lax.axis_index("s")
        c_idx = lax.axis_index("c")

        def _chunk(i, carry):
            cbase = c_idx * E + (s_idx + i * _NS) * _C
            sl = pl.ds(cbase, _C)
            pltpu.sync_copy(dstg_hbm.at[sl], dst_v)
            pltpu.sync_copy(srcg_hbm.at[sl], src_v)
            cp1 = pltpu.async_copy(td_hbm.at[dst_v], bufd, sem)
            cp2 = pltpu.async_copy(ts_hbm.at[src_v], bufs, sem2)
            cp1.wait()
            cp2.wait()
            pltpu.sync_copy(bufd, gd_hbm.at[sl])
            pltpu.sync_copy(bufs, gs_hbm.at[sl])
            return carry

        lax.fori_loop(0, per_tile, _chunk, 0)

    return k(td, ts, dstg, srcg)


def _gate(gd, gs, ea, we, eb):
    R, H2 = gd.shape
    H = H2 // 2
    Ep = R // _NC
    BR = 8192
    nb = Ep // BR

    def body(gd_ref, gs_ref, ea_ref, we_ref, eb_ref, m_ref):
        et = (jnp.dot(ea_ref[...], we_ref[0],
                      preferred_element_type=jnp.float32) + eb_ref[0])
        z = gd_ref[...] + gs_ref[...] + et
        f = z[:, :H]
        s = z[:, H:]
        sp = jnp.maximum(s, 0.0) + jnp.log1p(jnp.exp(-jnp.abs(s)))
        m_ref[...] = jax.nn.sigmoid(f) * sp

    return pl.pallas_call(
        body,
        grid=(_NC * nb,),
        in_specs=[pl.BlockSpec((BR, H2), lambda i: (i, 0)),
                  pl.BlockSpec((BR, H2), lambda i: (i, 0)),
                  pl.BlockSpec((BR, ea.shape[1]), lambda i: (i % nb, 0)),
                  pl.BlockSpec((1,) + we.shape[1:], lambda i: (i // nb, 0, 0)),
                  pl.BlockSpec((1, 1, H2), lambda i: (i // nb, 0, 0))],
        out_specs=pl.BlockSpec((BR, H), lambda i: (i, 0)),
        out_shape=jax.ShapeDtypeStruct((R, H), jnp.float32),
    )(gd, gs, ea, we, eb)


def _node_tables(x, wd, ws):
    NP, D = x.shape
    BN = 2048
    H2 = wd.shape[2]

    def body(x_ref, wd_ref, ws_ref, td_ref, ts_ref):
        xb = x_ref[...]
        for c in range(_NC):
            td_ref[c] = jnp.dot(xb, wd_ref[c],
                                preferred_element_type=jnp.float32)
            ts_ref[c] = jnp.dot(xb, ws_ref[c],
                                preferred_element_type=jnp.float32)

    ospec = pl.BlockSpec((_NC, BN, H2), lambda i: (0, i, 0))
    return pl.pallas_call(
        body,
        grid=(NP // BN,),
        in_specs=[pl.BlockSpec((BN, D), lambda i: (i, 0)),
                  pl.BlockSpec(wd.shape, lambda i: (0, 0, 0)),
                  pl.BlockSpec(ws.shape, lambda i: (0, 0, 0))],
        out_specs=[ospec] * 2,
        out_shape=[jax.ShapeDtypeStruct((_NC, NP, H2), jnp.float32)] * 2,
    )(x, wd, ws)


def _mid(x, p, wd, ws):
    NP, D = x.shape
    BN = 2048
    H2 = wd.shape[2]
    H = p.shape[2]

    def body(x_ref, p_ref, wd_ref, ws_ref, y_ref, td_ref, ts_ref):
        agg = jnp.concatenate([p_ref[0], p_ref[1]], axis=1)
        y = jax.nn.silu(x_ref[...] + agg)
        y_ref[...] = y
        for c in range(_NC):
            td_ref[c] = jnp.dot(y, wd_ref[c],
                                preferred_element_type=jnp.float32)
            ts_ref[c] = jnp.dot(y, ws_ref[c],
                                preferred_element_type=jnp.float32)

    ospec = pl.BlockSpec((_NC, BN, H2), lambda i: (0, i, 0))
    return pl.pallas_call(
        body,
        grid=(NP // BN,),
        in_specs=[pl.BlockSpec((BN, D), lambda i: (i, 0)),
                  pl.BlockSpec((_NC, BN, H), lambda i: (0, i, 0)),
                  pl.BlockSpec(wd.shape, lambda i: (0, 0, 0)),
                  pl.BlockSpec(ws.shape, lambda i: (0, 0, 0))],
        out_specs=[pl.BlockSpec((BN, D), lambda i: (i, 0)),
                   ospec, ospec],
        out_shape=[jax.ShapeDtypeStruct((NP, D), jnp.float32),
                   jax.ShapeDtypeStruct((_NC, NP, H2), jnp.float32),
                   jax.ShapeDtypeStruct((_NC, NP, H2), jnp.float32)],
    )(x, p, wd, ws)


def _head(x, p, n_nodes, w1, b1, w2a, w2b, b2, w3, b3, w4, b4, w5, b5,
          w6, b6, w7, b7, g):
    N = n_nodes

    def body(x_ref, p_ref, w1_ref, b1_ref, w2a_ref, w2b_ref, b2_ref,
             w3_ref, b3_ref, w4_ref, b4_ref, w5_ref, b5_ref, w6_ref,
             b6_ref, w7_ref, b7_ref, g_ref, o_ref):
        agg = jnp.concatenate([p_ref[0], p_ref[1]], axis=1)
        x3 = jax.nn.silu(x_ref[...] + agg)
        t = jax.nn.silu(jnp.dot(x3, w1_ref[...],
                                preferred_element_type=jnp.float32)
                        + b1_ref[...])
        t = jax.nn.silu(jnp.dot(t, w1_ref[...],
                                preferred_element_type=jnp.float32)
                        + b1_ref[...])
        pooled = jnp.sum(t[:N], axis=0, keepdims=True) * jnp.float32(1.0 / N)
        h = jax.nn.silu(
            jnp.dot(pooled, w2a_ref[...], preferred_element_type=jnp.float32)
            + jnp.dot(g_ref[...], w2b_ref[...],
                      preferred_element_type=jnp.float32)
            + b2_ref[...])
        for w_ref, b_ref in ((w3_ref, b3_ref), (w4_ref, b4_ref),
                             (w5_ref, b5_ref), (w6_ref, b6_ref)):
            h = jax.nn.silu(jnp.dot(h, w_ref[...],
                                    preferred_element_type=jnp.float32)
                            + b_ref[...])
        o_ref[...] = (jnp.dot(h, w7_ref[...],
                              preferred_element_type=jnp.float32)
                      + b7_ref[...])

    args = (x, p, w1, b1, w2a, w2b, b2, w3, b3, w4, b4, w5, b5, w6, b6,
            w7, b7, g)
    specs = [pl.BlockSpec(a.shape, lambda i, n=a.ndim: (0,) * n)
             for a in args]
    return pl.pallas_call(
        body,
        grid=(1,),
        in_specs=specs,
        out_specs=pl.BlockSpec((1, 1), lambda i: (0, 0)),
        out_shape=jax.ShapeDtypeStruct((1, 1), jnp.float32),
    )(*args)


def kernel(x, edge_index, edge_attr, glob_attr, batch,
           Wf1, bf1, Ws1, bs1, Wf2, bf2, Ws2, bs2, Wf3, bf3, Ws3, bs3,
           W1, b1, W2, b2, W3, b3, W4, b4, W5, b5, W6, b6, W7, b7):
    N, D = x.shape
    E = edge_index.shape[1]
    H = D // _NC
    npad = ((N + 128 * _NS - 1) // (128 * _NS)) * 128 * _NS
    egrp = 4096
    epad = ((E + egrp - 1) // egrp) * egrp

    xp = jnp.pad(x, ((0, npad - N), (0, 0)))
    src = jnp.pad(edge_index[0], (0, epad - E), constant_values=npad - 1)
    dst = jnp.pad(edge_index[1], (0, epad - E), constant_values=npad - 1)
    eap = jnp.pad(edge_attr, ((0, epad - E), (0, 0)))
    dstg = jnp.concatenate([dst, dst + npad])
    srcg = jnp.concatenate([src, src + npad])

    def wsplit(Wf, Ws, bf, bs):
        def cat(rows):
            return jnp.stack([
                jnp.concatenate([Wf[rows, c * H:(c + 1) * H],
                                 Ws[rows, c * H:(c + 1) * H]], axis=1)
                for c in range(_NC)])
        wd = cat(slice(0, D))
        wsrc = cat(slice(D, 2 * D))
        we = cat(slice(2 * D, None))
        eb = jnp.stack([
            jnp.concatenate([bf[c * H:(c + 1) * H],
                             bs[c * H:(c + 1) * H]])[None, :]
            for c in range(_NC)])
        return wd, wsrc, we, eb

    wd1, wsrc1, we1, eb1 = wsplit(Wf1, Ws1, bf1, bs1)
    wd2, wsrc2, we2, eb2 = wsplit(Wf2, Ws2, bf2, bs2)
    wd3, wsrc3, we3, eb3 = wsplit(Wf3, Ws3, bf3, bs3)

    def conv(td, ts, we, eb):
        gd, gs = _sc_gather(td.reshape(_NC * npad, 2 * H),
                            ts.reshape(_NC * npad, 2 * H), dstg, srcg)
        m = _gate(gd, gs, eap, we, eb)
        return jnp.stack(
            [jax.ops.segment_sum(m[c * epad:(c + 1) * epad], dst,
                                 num_segments=npad)
             for c in range(_NC)])

    td, ts = _node_tables(xp, wd1, wsrc1)
    p = conv(td, ts, we1, eb1)
    x1, td, ts = _mid(xp, p, wd2, wsrc2)
    p = conv(td, ts, we2, eb2)
    x2, td, ts = _mid(x1, p, wd3, wsrc3)
    p = conv(td, ts, we3, eb3)

    w2a = W2[:D]
    w2b = W2[D:]
    return _head(x2, p, N, W1, b1[None, :], w2a, w2b, b2[None, :],
                 W3, b3[None, :], W4, b4[None, :], W5, b5[None, :],
                 W6, b6[None, :], W7, b7[None, :], glob_attr.reshape(1, -1))
